# baseline (speedup 1.0000x reference)
import functools

import jax
import jax.numpy as jnp
from jax import lax
from jax.experimental import pallas as pl
from jax.experimental.pallas import tpu as pltpu
from jax.experimental.pallas import tpu_sc as plsc

N_TRAIN = 100000
FEAT = 27
N_APP = 10
K = 3
BIGF = 3.0e38

N_SUB = 16
N_PAD = 100352
CHUNK = N_PAD // N_SUB
NVEC = CHUNK // 16


def _dist_body(q_ref, xt_ref, d2_ref):
    xt = xt_ref[...]
    q = q_ref[...]
    scale = jnp.max(jnp.abs(xt), axis=1, keepdims=True)
    inv = jnp.where(scale != 0.0, 1.0 / scale, 0.0)
    qs = q * inv
    diff = xt * inv - qs
    d2 = jnp.sum(diff * diff, axis=0, keepdims=True)
    pad = jnp.full((1, N_PAD - N_TRAIN), BIGF, jnp.float32)
    d2_ref[...] = jnp.concatenate([d2, pad], axis=1).reshape((N_PAD,))


def _dist_tc(input_tensor, feats_t, interpret=False):
    return pl.pallas_call(
        _dist_body,
        out_shape=jax.ShapeDtypeStruct((N_PAD,), jnp.float32),
        interpret=interpret,
    )(input_tensor, feats_t)


def _fsqrt(a):
    bits = lax.bitcast_convert_type(a, jnp.int32)
    y = lax.bitcast_convert_type((bits >> 1) + jnp.int32(0x1FBD1DF5), jnp.float32)
    for _ in range(4):
        y = 0.5 * (y + a / y)
    return jnp.where(a == 0.0, 0.0, y)


def _topk_body(d2_hbm, labf_hbm, dist_out, res_out, cand_hbm,
               d2_v, stage_d, m_cand,
               lab_v, out_d, out_r, sem):
    c = lax.axis_index("c")
    s = lax.axis_index("s")

    @pl.when(c == 0)
    def _work():
        base = s * CHUNK
        pltpu.sync_copy(d2_hbm.at[pl.ds(base, CHUNK)], d2_v)
        iota = lax.iota(jnp.int32, 16)
        inf = jnp.float32(BIGF)
        bigi = jnp.int32(2**31 - 1)

        finf = jnp.full((16,), BIGF, jnp.float32)
        fbig = jnp.full((16,), 2**31 - 1, jnp.int32)

        UNROLL = 8

        def step(u, carry):
            m1, i1, m2, i2, m3, i3 = carry
            for uu in range(UNROLL):
                j = u * UNROLL + uu
                v = d2_v[pl.ds(j * 16, 16)]
                gi = base + j * 16 + iota
                b1 = v < m1
                b2 = v < m2
                b3 = v < m3
                m3n = jnp.where(b2, m2, jnp.where(b3, v, m3))
                i3n = jnp.where(b2, i2, jnp.where(b3, gi, i3))
                m2n = jnp.where(b1, m1, jnp.where(b2, v, m2))
                i2n = jnp.where(b1, i1, jnp.where(b2, gi, i2))
                m1, i1 = jnp.where(b1, v, m1), jnp.where(b1, gi, i1)
                m2, i2, m3, i3 = m2n, i2n, m3n, i3n
            return m1, i1, m2, i2, m3, i3

        m1, i1, m2, i2, m3, i3 = lax.fori_loop(
            0, NVEC // UNROLL, step, (finf, fbig, finf, fbig, finf, fbig))

        excl = []
        tops = []
        for _ in range(K):
            a1, a2, a3 = m1, m2, m3
            for e in excl:
                a1 = jnp.where(i1 == e, inf, a1)
                a2 = jnp.where(i2 == e, inf, a2)
                a3 = jnp.where(i3 == e, inf, a3)
            mm = jnp.minimum(jnp.minimum(a1, a2), a3)
            mval = jnp.min(mm)
            cand = jnp.where(a1 == mval, i1, fbig)
            cand = jnp.minimum(cand, jnp.where(a2 == mval, i2, fbig))
            cand = jnp.minimum(cand, jnp.where(a3 == mval, i3, fbig))
            midx = jnp.min(cand)
            tops.append((mval, midx))
            excl.append(midx)

        ib = [lax.bitcast_convert_type(tops[k][1], jnp.float32)
              for k in range(K)]
        packed = jnp.where(iota == 0, tops[0][0],
                 jnp.where(iota == 1, tops[1][0],
                 jnp.where(iota == 2, tops[2][0],
                 jnp.where(iota == 3, ib[0],
                 jnp.where(iota == 4, ib[1],
                 jnp.where(iota == 5, ib[2], inf))))))
        stage_d[...] = packed
        pltpu.sync_copy(stage_d, cand_hbm.at[s])
        plsc.subcore_barrier()

        @pl.when(s == 0)
        def _merge():
            pltpu.sync_copy(cand_hbm, m_cand)
            shift3 = jnp.minimum(iota + 3, 15)
            rows_d = []
            rows_i = []
            for r in range(N_SUB):
                row = m_cand[r]
                rows_d.append(jnp.where(iota < K, row, inf))
                bits = plsc.load_gather(
                    m_cand, [jnp.full((16,), r, jnp.int32), shift3])
                ridx = lax.bitcast_convert_type(bits, jnp.int32)
                rows_i.append(jnp.where(iota < K, ridx, bigi))

            excl2 = []
            glob = []
            for _ in range(K):
                vmin, vidx = finf, fbig
                for r in range(N_SUB):
                    row, ridx = rows_d[r], rows_i[r]
                    for e in excl2:
                        row = jnp.where(ridx == e, inf, row)
                    b = row < vmin
                    vmin = jnp.where(b, row, vmin)
                    vidx = jnp.where(b, ridx, vidx)
                mval = jnp.min(vmin)
                midx = jnp.min(jnp.where(vmin == mval, vidx, fbig))
                glob.append((mval, midx))
                excl2.append(midx)

            md = [g[0] for g in glob]
            mi = [g[1] for g in glob]
            d2vec = jnp.where(iota == 0, md[0],
                    jnp.where(iota == 1, md[1],
                    jnp.where(iota == 2, md[2], 1.0)))
            dvec = _fsqrt(d2vec)
            dsafe = jnp.where(dvec == 0.0, 1.0, dvec)
            wvec = jnp.where(iota < K, 1.0 / dsafe, 0.0)
            denom = jnp.sum(wvec)
            wnorm = wvec / denom
            exact = md[0] == 0.0
            onehot0 = jnp.where(iota == 0, 1.0, 0.0)
            avec = jnp.where(exact, onehot0, wnorm)

            app = jnp.minimum(iota, N_APP - 1)
            copies = []
            for k in range(K):
                nk = jnp.minimum(jnp.maximum(mi[k], 0), N_TRAIN - 1)
                idxg = app * N_TRAIN + nk
                copies.append(
                    pltpu.async_copy(labf_hbm.at[idxg], lab_v.at[k], sem))
            for cp in copies:
                cp.wait()

            lanes_ok = iota < N_APP
            res = jnp.zeros((16,), jnp.float32)
            for k in range(K):
                lk = jnp.where(lanes_ok, lab_v[k], 0.0)
                ak = jnp.min(jnp.where(iota == k, avec, BIGF))
                res = res + ak * lk

            out_d[...] = jnp.where(iota < K, dvec, 0.0)
            out_r[...] = res
            pltpu.sync_copy(out_d, dist_out)
            pltpu.sync_copy(out_r, res_out)


def _topk_sc(d2_pad, labels_flat):
    mesh = plsc.VectorSubcoreMesh(core_axis_name="c", subcore_axis_name="s")
    f = pl.kernel(
        _topk_body,
        out_type=(
            jax.ShapeDtypeStruct((16,), jnp.float32),
            jax.ShapeDtypeStruct((16,), jnp.float32),
            jax.ShapeDtypeStruct((N_SUB, 16), jnp.float32),
        ),
        mesh=mesh,
        compiler_params=pltpu.CompilerParams(needs_layout_passes=False),
        scratch_types=[
            pltpu.VMEM((CHUNK,), jnp.float32),
            pltpu.VMEM((16,), jnp.float32),
            pltpu.VMEM((N_SUB, 16), jnp.float32),
            pltpu.VMEM((K, 16), jnp.float32),
            pltpu.VMEM((16,), jnp.float32),
            pltpu.VMEM((16,), jnp.float32),
            pltpu.SemaphoreType.DMA,
        ],
    )
    dist, res, _cand = f(d2_pad, labels_flat)
    return dist, res


def kernel(input_tensor, training_data_features, training_data_labels):
    feats_t = training_data_features.T
    labels_flat = training_data_labels.T.reshape(-1)
    d2_pad = _dist_tc(input_tensor, feats_t)
    dist, res = _topk_sc(d2_pad, labels_flat)
    return dist[:K], res[:N_APP]


# speedup vs baseline: 1.0060x; 1.0060x over previous
import jax
import jax.numpy as jnp
from jax import lax
from jax.experimental import pallas as pl
from jax.experimental.pallas import tpu as pltpu
from jax.experimental.pallas import tpu_sc as plsc

N_TRAIN = 100000
FEAT = 27
N_APP = 10
K = 3
BIGF = 3.0e38

N_SUB = 16
N_PAD = 100352
CHUNK = N_PAD // N_SUB
NVEC = CHUNK // 16


def _dist_body(q_ref, xt_ref, d2_ref):
    xt = xt_ref[...]
    q = q_ref[...]
    scale = jnp.max(jnp.abs(xt), axis=1, keepdims=True)
    inv = jnp.where(scale != 0.0, 1.0 / scale, 0.0)
    qs = q * inv
    diff = xt * inv - qs
    d2 = jnp.sum(diff * diff, axis=0, keepdims=True)
    pad = jnp.full((1, N_PAD - N_TRAIN), BIGF, jnp.float32)
    d2_ref[...] = jnp.concatenate([d2, pad], axis=1).reshape((N_PAD,))


def _dist_tc(input_tensor, feats_t):
    return pl.pallas_call(
        _dist_body,
        out_shape=jax.ShapeDtypeStruct((N_PAD,), jnp.float32),
    )(input_tensor, feats_t)


def _fsqrt(a):
    bits = lax.bitcast_convert_type(a, jnp.int32)
    y = lax.bitcast_convert_type((bits >> 1) + jnp.int32(0x1FBD1DF5), jnp.float32)
    for _ in range(4):
        y = 0.5 * (y + a / y)
    return jnp.where(a == 0.0, 0.0, y)


def _topk_body(d2_hbm, labf_hbm, dist_out, res_out, cand_hbm,
               d2_v, stage_d, m_cand,
               lab_v, out_d, out_r, sem):
    c = lax.axis_index("c")
    s = lax.axis_index("s")

    @pl.when(c == 0)
    def _work():
        base = s * CHUNK
        pltpu.sync_copy(d2_hbm.at[pl.ds(base, CHUNK)], d2_v)
        iota = lax.iota(jnp.int32, 16)
        inf = jnp.float32(BIGF)
        bigi = jnp.int32(2**31 - 1)

        finf = jnp.full((16,), BIGF, jnp.float32)
        fbig = jnp.full((16,), 2**31 - 1, jnp.int32)

        UNROLL = 8

        def step(u, carry):
            m1, i1, m2, i2, m3, i3 = carry
            for uu in range(UNROLL):
                j = u * UNROLL + uu
                v = d2_v[pl.ds(j * 16, 16)]
                gi = base + j * 16 + iota
                b1 = v < m1
                b2 = v < m2
                b3 = v < m3
                m3n = jnp.where(b2, m2, jnp.where(b3, v, m3))
                i3n = jnp.where(b2, i2, jnp.where(b3, gi, i3))
                m2n = jnp.where(b1, m1, jnp.where(b2, v, m2))
                i2n = jnp.where(b1, i1, jnp.where(b2, gi, i2))
                m1, i1 = jnp.where(b1, v, m1), jnp.where(b1, gi, i1)
                m2, i2, m3, i3 = m2n, i2n, m3n, i3n
            return m1, i1, m2, i2, m3, i3

        m1, i1, m2, i2, m3, i3 = lax.fori_loop(
            0, NVEC // UNROLL, step, (finf, fbig, finf, fbig, finf, fbig))

        excl = []
        tops = []
        for _ in range(K):
            a1, a2, a3 = m1, m2, m3
            for e in excl:
                a1 = jnp.where(i1 == e, inf, a1)
                a2 = jnp.where(i2 == e, inf, a2)
                a3 = jnp.where(i3 == e, inf, a3)
            mm = jnp.minimum(jnp.minimum(a1, a2), a3)
            mval = jnp.min(mm)
            cand = jnp.where(a1 == mval, i1, fbig)
            cand = jnp.minimum(cand, jnp.where(a2 == mval, i2, fbig))
            cand = jnp.minimum(cand, jnp.where(a3 == mval, i3, fbig))
            midx = jnp.min(cand)
            tops.append((mval, midx))
            excl.append(midx)

        ib = [lax.bitcast_convert_type(tops[k][1], jnp.float32)
              for k in range(K)]
        packed = jnp.where(iota == 0, tops[0][0],
                 jnp.where(iota == 1, tops[1][0],
                 jnp.where(iota == 2, tops[2][0],
                 jnp.where(iota == 3, ib[0],
                 jnp.where(iota == 4, ib[1],
                 jnp.where(iota == 5, ib[2], inf))))))
        stage_d[...] = packed
        pltpu.sync_copy(stage_d, cand_hbm.at[s])
        plsc.subcore_barrier()

        @pl.when(s == 0)
        def _merge():
            pltpu.sync_copy(cand_hbm, m_cand)
            shift3 = jnp.minimum(iota + 3, 15)
            rows_d = []
            rows_i = []
            for r in range(N_SUB):
                row = m_cand[r]
                rows_d.append(jnp.where(iota < K, row, inf))
                bits = plsc.load_gather(
                    m_cand, [jnp.full((16,), r, jnp.int32), shift3])
                ridx = lax.bitcast_convert_type(bits, jnp.int32)
                rows_i.append(jnp.where(iota < K, ridx, bigi))

            excl2 = []
            glob = []
            for _ in range(K):
                vmin, vidx = finf, fbig
                for r in range(N_SUB):
                    row, ridx = rows_d[r], rows_i[r]
                    for e in excl2:
                        row = jnp.where(ridx == e, inf, row)
                    b = row < vmin
                    vmin = jnp.where(b, row, vmin)
                    vidx = jnp.where(b, ridx, vidx)
                mval = jnp.min(vmin)
                midx = jnp.min(jnp.where(vmin == mval, vidx, fbig))
                glob.append((mval, midx))
                excl2.append(midx)

            md = [g[0] for g in glob]
            mi = [g[1] for g in glob]
            d2vec = jnp.where(iota == 0, md[0],
                    jnp.where(iota == 1, md[1],
                    jnp.where(iota == 2, md[2], 1.0)))
            dvec = _fsqrt(d2vec)
            dsafe = jnp.where(dvec == 0.0, 1.0, dvec)
            wvec = jnp.where(iota < K, 1.0 / dsafe, 0.0)
            denom = jnp.sum(wvec)
            wnorm = wvec / denom
            exact = md[0] == 0.0
            onehot0 = jnp.where(iota == 0, 1.0, 0.0)
            avec = jnp.where(exact, onehot0, wnorm)

            app = jnp.minimum(iota, N_APP - 1)
            copies = []
            for k in range(K):
                nk = jnp.minimum(jnp.maximum(mi[k], 0), N_TRAIN - 1)
                idxg = app * N_TRAIN + nk
                copies.append(
                    pltpu.async_copy(labf_hbm.at[idxg], lab_v.at[k], sem))
            for cp in copies:
                cp.wait()

            lanes_ok = iota < N_APP
            res = jnp.zeros((16,), jnp.float32)
            for k in range(K):
                lk = jnp.where(lanes_ok, lab_v[k], 0.0)
                ak = jnp.min(jnp.where(iota == k, avec, BIGF))
                res = res + ak * lk

            out_d[...] = jnp.where(iota < K, dvec, 0.0)
            out_r[...] = res
            pltpu.sync_copy(out_d, dist_out)
            pltpu.sync_copy(out_r, res_out)


def _topk_sc(d2_pad, labels_flat):
    mesh = plsc.VectorSubcoreMesh(core_axis_name="c", subcore_axis_name="s")
    f = pl.kernel(
        _topk_body,
        out_type=(
            jax.ShapeDtypeStruct((16,), jnp.float32),
            jax.ShapeDtypeStruct((16,), jnp.float32),
            jax.ShapeDtypeStruct((N_SUB, 16), jnp.float32),
        ),
        mesh=mesh,
        compiler_params=pltpu.CompilerParams(needs_layout_passes=False),
        scratch_types=[
            pltpu.VMEM((CHUNK,), jnp.float32),
            pltpu.VMEM((16,), jnp.float32),
            pltpu.VMEM((N_SUB, 16), jnp.float32),
            pltpu.VMEM((K, 16), jnp.float32),
            pltpu.VMEM((16,), jnp.float32),
            pltpu.VMEM((16,), jnp.float32),
            pltpu.SemaphoreType.DMA,
        ],
    )
    dist, res, _cand = f(d2_pad, labels_flat)
    return dist, res


def kernel(input_tensor, training_data_features, training_data_labels):
    feats_t = training_data_features.T
    labels_flat = training_data_labels.T.reshape(-1)
    d2_pad = _dist_tc(input_tensor, feats_t)
    dist, res = _topk_sc(d2_pad, labels_flat)
    return dist[:K], res[:N_APP]
